# baseline (speedup 1.0000x reference)
import jax
import jax.numpy as jnp
from jax.experimental import pallas as pl
from jax.experimental.pallas import tpu as pltpu

_CHUNK = 256


def _split(v):
    hi = v.astype(jnp.bfloat16)
    lo = (v - hi.astype(jnp.float32)).astype(jnp.bfloat16)
    return hi, lo


def _dot(a, b):
    return jax.lax.dot_general(
        a, b, (((1,), (0,)), ((), ())), preferred_element_type=jnp.float32
    )


def _net_kernel(x_ref, adj_hbm, W1a_ref, b1a_ref, W1b_ref, b1b_ref,
                W2a_ref, b2a_ref, W2b_ref, b2b_ref,
                W3a_ref, b3a_ref, W3b_ref, b3b_ref, out_ref,
                A_bf, buf, sem):
    N = adj_hbm.shape[0]
    n_chunks = N // _CHUNK
    g = x_ref[...].T

    def copy_chunk(k):
        return pltpu.make_async_copy(
            adj_hbm.at[pl.ds(k * _CHUNK, _CHUNK), :], buf.at[k % 2], sem.at[k % 2]
        )

    hi, lo = _split(g)
    copy_chunk(0).start()
    acc = jnp.zeros_like(g)
    for k in range(n_chunks):
        if k + 1 < n_chunks:
            copy_chunk(k + 1).start()
        copy_chunk(k).wait()
        a_k = buf[k % 2].astype(jnp.bfloat16)
        A_bf[pl.ds(k * _CHUNK, _CHUNK), :] = a_k
        sl = slice(k * _CHUNK, (k + 1) * _CHUNK)
        acc = acc + _dot(hi[:, sl], a_k) + _dot(lo[:, sl], a_k)

    def agg_dot(t):
        thi, tlo = _split(t)
        A = A_bf[...]
        return _dot(thi, A) + _dot(tlo, A)

    def mlp_dot(w_ref, t):
        w1, w2 = _split(w_ref[...].T)
        t1, t2 = _split(t)
        return _dot(w1, t1) + (_dot(w1, t2) + _dot(w2, t1))

    def mlp(z, Wa_ref, ba_ref, Wb_ref, bb_ref):
        u = jnp.maximum(mlp_dot(Wa_ref, z) + ba_ref[...].reshape(-1, 1), 0.0)
        return mlp_dot(Wb_ref, u) + bb_ref[...].reshape(-1, 1)

    g = jnp.maximum(mlp(g + acc, W1a_ref, b1a_ref, W1b_ref, b1b_ref), 0.0)
    g = jnp.maximum(mlp(g + agg_dot(g), W2a_ref, b2a_ref, W2b_ref, b2b_ref), 0.0)
    g = mlp(g + agg_dot(g), W3a_ref, b3a_ref, W3b_ref, b3b_ref)
    out_ref[...] = jnp.mean(g, axis=1, keepdims=True).T


@jax.jit
def kernel(x, adj, W1a, b1a, W1b, b1b, W2a, b2a, W2b, b2b, W3a, b3a, W3b, b3b):
    N = adj.shape[0]
    O = W3b.shape[1]
    vmem = pl.BlockSpec(memory_space=pltpu.MemorySpace.VMEM)
    return pl.pallas_call(
        _net_kernel,
        out_shape=jax.ShapeDtypeStruct((1, O), jnp.float32),
        in_specs=[vmem, pl.BlockSpec(memory_space=pltpu.MemorySpace.HBM)]
        + [vmem] * 12,
        scratch_shapes=[
            pltpu.VMEM((N, N), jnp.bfloat16),
            pltpu.VMEM((2, _CHUNK, N), jnp.float32),
            pltpu.SemaphoreType.DMA((2,)),
        ],
        compiler_params=pltpu.CompilerParams(
            vmem_limit_bytes=100 * 1024 * 1024,
        ),
    )(x, adj, W1a, b1a, W1b, b1b, W2a, b2a, W2b, b2b, W3a, b3a, W3b, b3b)


# speedup vs baseline: 1.1346x; 1.1346x over previous
import jax
import jax.numpy as jnp
from jax.experimental import pallas as pl
from jax.experimental.pallas import tpu as pltpu

_CHUNK = 256


def _split(v):
    hi = v.astype(jnp.bfloat16)
    lo = (v - hi.astype(jnp.float32)).astype(jnp.bfloat16)
    return hi, lo


def _dot(a, b):
    return jax.lax.dot_general(
        a, b, (((1,), (0,)), ((), ())), preferred_element_type=jnp.float32
    )


def _net_kernel(x_ref, adj_hbm, W1a_ref, b1a_ref, W1b_ref, b1b_ref,
                W2a_ref, b2a_ref, W2b_ref, b2b_ref,
                W3a_ref, b3a_ref, W3b_ref, b3b_ref, out_ref,
                A_f32, A_bf, sem):
    N = adj_hbm.shape[0]
    F = x_ref.shape[1]
    n_chunks = N // _CHUNK

    def chunk_copy(k):
        sl = pl.ds(k * _CHUNK, _CHUNK)
        return pltpu.make_async_copy(adj_hbm.at[sl, :], A_f32.at[sl, :], sem.at[k])

    for k in range(n_chunks):
        chunk_copy(k).start()

    def mlp_dot(w_ref, t):
        w1, w2 = _split(w_ref[...].T)
        t1, t2 = _split(t)
        p = _dot(jnp.concatenate([w1, w2], axis=0), t1)
        return p[:F] + p[F:] + _dot(w1, t2)

    def mlp(z, Wa_ref, ba_ref, Wb_ref, bb_ref):
        u = jnp.maximum(mlp_dot(Wa_ref, z) + ba_ref[...].reshape(-1, 1), 0.0)
        return mlp_dot(Wb_ref, u) + bb_ref[...].reshape(-1, 1)

    g = x_ref[...].T
    hi, lo = _split(g)
    s = jnp.concatenate([hi, lo], axis=0)

    acc2 = jnp.zeros((2 * F, N), jnp.float32)
    for k in range(n_chunks):
        chunk_copy(k).wait()
        sl = slice(k * _CHUNK, (k + 1) * _CHUNK)
        a_k = A_f32[sl, :].astype(jnp.bfloat16)
        A_bf[sl, :] = a_k
        acc2 = acc2 + _dot(s[:, sl], a_k)
    acc = acc2[:F] + acc2[F:]

    def agg_dot(t):
        thi, tlo = _split(t)
        r = _dot(jnp.concatenate([thi, tlo], axis=0), A_bf[...])
        return r[:F] + r[F:]

    g = jnp.maximum(mlp(g + acc, W1a_ref, b1a_ref, W1b_ref, b1b_ref), 0.0)
    g = jnp.maximum(mlp(g + agg_dot(g), W2a_ref, b2a_ref, W2b_ref, b2b_ref), 0.0)
    g = mlp(g + agg_dot(g), W3a_ref, b3a_ref, W3b_ref, b3b_ref)
    out_ref[...] = jnp.mean(g, axis=1, keepdims=True).T


@jax.jit
def kernel(x, adj, W1a, b1a, W1b, b1b, W2a, b2a, W2b, b2b, W3a, b3a, W3b, b3b):
    N = adj.shape[0]
    O = W3b.shape[1]
    vmem = pl.BlockSpec(memory_space=pltpu.MemorySpace.VMEM)
    return pl.pallas_call(
        _net_kernel,
        out_shape=jax.ShapeDtypeStruct((1, O), jnp.float32),
        in_specs=[vmem, pl.BlockSpec(memory_space=pltpu.MemorySpace.HBM)]
        + [vmem] * 12,
        scratch_shapes=[
            pltpu.VMEM((N, N), jnp.float32),
            pltpu.VMEM((N, N), jnp.bfloat16),
            pltpu.SemaphoreType.DMA((N // _CHUNK,)),
        ],
        compiler_params=pltpu.CompilerParams(
            vmem_limit_bytes=100 * 1024 * 1024,
        ),
    )(x, adj, W1a, b1a, W1b, b1b, W2a, b2a, W2b, b2b, W3a, b3a, W3b, b3b)
